# baseline (speedup 1.0000x reference)
import jax
import jax.numpy as jnp
from jax import lax
from jax.experimental import pallas as pl
from jax.experimental.pallas import tpu as pltpu
from jax.experimental.pallas import tpu_sc as plsc

_N = 100128
_G = 448
_D = 128
_NW = 32
_RPW = 3136
_CH = 224
_NCH = _RPW // _CH
_NCHL = (_N - (_NW - 1) * _RPW) // _CH
_BPAD = _G + 16
_BIG = 2**30
_M0 = 0
_M1 = _BPAD
_M2 = 2 * _BPAD
_M3 = 2 * _BPAD + _NW
_MLEN = 2 * _BPAD + _NW + _NW * _NCH


def _sc_body(nodes_hbm, meta_hbm, out_hbm, meta_v, buf_v, acc_v, sem, meta_s):
    wid = lax.axis_index("s") * 2 + lax.axis_index("c")
    r0 = wid * _RPW

    pltpu.sync_copy(meta_hbm, meta_v)

    def stage(i, _):
        v = meta_v[pl.ds(i * 16, 16)]
        for j in range(16):
            meta_s[i * 16 + j] = v[j]
        return 0
    lax.fori_loop(0, _MLEN // 16, stage, 0)

    def zbody(i, _):
        for j in range(8):
            acc_v[i, pl.ds(j * 16, 16)] = jnp.zeros((16,), jnp.float32)
        return 0
    lax.fori_loop(0, _G, zbody, 0)

    def chunk_body(c, k):
        cs = r0 + c * _CH
        ce = cs + _CH
        pltpu.sync_copy(nodes_hbm.at[pl.ds(cs, _CH)], buf_v)
        nseg = meta_s[_M3 + wid * _NCH + c]

        def seg_body(t, k):
            lo = jnp.maximum(meta_s[_M0 + k], cs)
            hi = jnp.minimum(meta_s[_M0 + k + 1], ce)
            g = meta_s[_M1 + k]

            def row_body(row, s):
                base = row - cs
                return tuple(s[j] + buf_v[base, pl.ds(j * 16, 16)]
                             for j in range(8))

            s = lax.fori_loop(
                lo, hi, row_body,
                tuple(jnp.zeros((16,), jnp.float32) for _ in range(8)))
            for j in range(8):
                acc_v[g, pl.ds(j * 16, 16)] = (
                    acc_v[g, pl.ds(j * 16, 16)] + s[j])
            return jnp.where(meta_s[_M0 + k + 1] <= ce, k + 1, k)

        return lax.fori_loop(0, nseg, seg_body, k)

    nch = jnp.where(wid == _NW - 1, _NCHL, _NCH)
    lax.fori_loop(0, nch, chunk_body, meta_s[_M2 + wid])
    pltpu.sync_copy(acc_v, out_hbm.at[wid])


def _sc_segment_sum(nodes, meta):
    mesh = plsc.VectorSubcoreMesh(core_axis_name="c", subcore_axis_name="s")
    return pl.kernel(
        _sc_body,
        out_type=jax.ShapeDtypeStruct((_NW, _G, _D), jnp.float32),
        mesh=mesh,
        scratch_types=[
            pltpu.VMEM((_MLEN,), jnp.int32),
            pltpu.VMEM((_CH, _D), jnp.float32),
            pltpu.VMEM((_G, _D), jnp.float32),
            pltpu.SemaphoreType.DMA,
            pltpu.SMEM((_MLEN,), jnp.int32),
        ],
    )(nodes, meta)


def _reduce_mlp_kernel(p_ref, w1_ref, b1_ref, w2_ref, b2_ref, out_ref):
    s = jnp.sum(p_ref[...], axis=0)
    h = jnp.dot(s, w1_ref[...], preferred_element_type=jnp.float32)
    h = jnp.maximum(h + b1_ref[...], 0.0)
    o = jnp.dot(h, w2_ref[...], preferred_element_type=jnp.float32)
    out_ref[...] = o + b2_ref[...]


def _build_meta(graphs_size):
    sizes = graphs_size.astype(jnp.int32)
    off = jnp.concatenate([jnp.zeros((1,), jnp.int32),
                           jnp.cumsum(sizes, dtype=jnp.int32)])
    starts = jnp.where(sizes > 0, off[:_G], jnp.int32(_BIG))
    order = jnp.argsort(starts, stable=True)
    bstart = jnp.concatenate(
        [starts[order], jnp.full((_BPAD - _G,), _BIG, jnp.int32)])
    bgraph = jnp.concatenate(
        [order.astype(jnp.int32), jnp.zeros((_BPAD - _G,), jnp.int32)])
    r0s = jnp.arange(_NW, dtype=jnp.int32) * _RPW
    cs = r0s[:, None] + jnp.arange(_NCH, dtype=jnp.int32)[None, :] * _CH
    ks = jnp.searchsorted(bstart, cs, side="right").astype(jnp.int32) - 1
    ke = jnp.searchsorted(bstart, cs + _CH - 1,
                          side="right").astype(jnp.int32) - 1
    nseg = ke - ks + 1
    return jnp.concatenate(
        [bstart, bgraph, ks[:, 0], nseg.reshape(-1)])


def kernel(nodes_embedding, graphs_size, W1, b1, W2, b2):
    meta = _build_meta(graphs_size)
    partials = _sc_segment_sum(nodes_embedding, meta)
    out = pl.pallas_call(
        _reduce_mlp_kernel,
        out_shape=jax.ShapeDtypeStruct((_G, b2.shape[0]), jnp.float32),
    )(partials, W1, b1.reshape(1, -1), W2, b2.reshape(1, -1))
    return out


# speedup vs baseline: 2.5653x; 2.5653x over previous
import jax
import jax.numpy as jnp
from jax import lax
from jax.experimental import pallas as pl
from jax.experimental.pallas import tpu as pltpu
from jax.experimental.pallas import tpu_sc as plsc

_N = 100128
_G = 448
_D = 128
_NW = 32
_RPW = 3136
_CH = 224
_NCH = _RPW // _CH
_OFFPAD = 464
_M0 = 0
_M1 = _OFFPAD
_M2 = _OFFPAD + _NW
_MLEN = _OFFPAD + _NW + _NW * _NCH


def _sc_body(nodes_hbm, meta_hbm, out_hbm,
             meta_v, buf0, buf1, acc_v, sem0, sem1, meta_s):
    wid = lax.axis_index("s") * 2 + lax.axis_index("c")
    r0 = wid * _RPW

    pltpu.sync_copy(meta_hbm, meta_v)

    def stage(i, _):
        v = meta_v[pl.ds(i * 16, 16)]
        for j in range(16):
            meta_s[i * 16 + j] = v[j]
        return 0
    lax.fori_loop(0, _MLEN // 16, stage, 0)

    def zbody(i, _):
        for j in range(8):
            acc_v[i, pl.ds(j * 16, 16)] = jnp.zeros((16,), jnp.float32)
        return 0
    lax.fori_loop(0, _G, zbody, 0)

    bufs = (buf0, buf1)
    sems = (sem0, sem1)

    def copy(c, b):
        dstart = jnp.minimum(r0 + c * _CH, _N - _CH)
        return pltpu.make_async_copy(
            nodes_hbm.at[pl.ds(dstart, _CH)], bufs[b], sems[b])

    copy(0, 0).start()
    copy(1, 1).start()

    def process(c, buf, k):
        cs = r0 + c * _CH
        ce = cs + _CH
        nseg = meta_s[_M2 + wid * _NCH + c]

        def seg_body(t, k):
            lo = jnp.maximum(meta_s[_M0 + k], cs)
            hi = jnp.minimum(meta_s[_M0 + k + 1], ce)

            def row_body(row, s):
                base = row - cs
                return tuple(s[j] + buf[base, pl.ds(j * 16, 16)]
                             for j in range(8))

            s = lax.fori_loop(
                lo, hi, row_body,
                tuple(jnp.zeros((16,), jnp.float32) for _ in range(8)))
            for j in range(8):
                acc_v[k, pl.ds(j * 16, 16)] = (
                    acc_v[k, pl.ds(j * 16, 16)] + s[j])
            return jnp.where(meta_s[_M0 + k + 1] <= ce, k + 1, k)

        return lax.fori_loop(0, nseg, seg_body, k)

    def pair_body(p, k):
        for b in range(2):
            c = 2 * p + b
            copy(c, b).wait()
            k = process(c, bufs[b], k)

            @pl.when(c + 2 < _NCH)
            def _():
                copy(c + 2, b).start()
        return k

    lax.fori_loop(0, _NCH // 2, pair_body, meta_s[_M1 + wid])
    pltpu.sync_copy(acc_v, out_hbm.at[wid])


def _sc_segment_sum(nodes, meta):
    mesh = plsc.VectorSubcoreMesh(core_axis_name="c", subcore_axis_name="s")
    return pl.kernel(
        _sc_body,
        out_type=jax.ShapeDtypeStruct((_NW, _G, _D), jnp.float32),
        mesh=mesh,
        scratch_types=[
            pltpu.VMEM((_MLEN,), jnp.int32),
            pltpu.VMEM((_CH, _D), jnp.float32),
            pltpu.VMEM((_CH, _D), jnp.float32),
            pltpu.VMEM((_G, _D), jnp.float32),
            pltpu.SemaphoreType.DMA,
            pltpu.SemaphoreType.DMA,
            pltpu.SMEM((_MLEN,), jnp.int32),
        ],
    )(nodes, meta)


def _reduce_mlp_kernel(p_ref, w1_ref, b1_ref, w2_ref, b2_ref, out_ref):
    s = jnp.sum(p_ref[...], axis=0)
    h = jnp.dot(s, w1_ref[...], preferred_element_type=jnp.float32)
    h = jnp.maximum(h + b1_ref[...], 0.0)
    o = jnp.dot(h, w2_ref[...], preferred_element_type=jnp.float32)
    out_ref[...] = o + b2_ref[...]


def _build_meta(graphs_size):
    sizes = graphs_size.astype(jnp.int32)
    off = jnp.concatenate([jnp.zeros((1,), jnp.int32),
                           jnp.cumsum(sizes, dtype=jnp.int32)])
    off_pad = jnp.concatenate(
        [off, jnp.full((_OFFPAD - _G - 1,), jnp.int32(_N))])

    def count_le(q):
        return jnp.sum((off[None, :] <= q[:, :, None]).astype(jnp.int32),
                       axis=-1)

    r0s = jnp.arange(_NW, dtype=jnp.int32)[:, None] * _RPW
    cs = r0s + jnp.arange(_NCH, dtype=jnp.int32)[None, :] * _CH
    k_last = count_le(cs + _CH - 1) - 1
    m = count_le(cs + _CH)
    k_in0 = count_le(r0s) - 1
    k_in = jnp.concatenate(
        [k_in0, k_last[:, :-1] + (k_last[:, :-1] + 2 <= m[:, :-1])], axis=1)
    nseg = k_last - k_in + 1
    nseg = jnp.where(cs < _N, nseg, 0)
    return jnp.concatenate(
        [off_pad, k_in[:, 0], nseg.reshape(-1)]).astype(jnp.int32)


def kernel(nodes_embedding, graphs_size, W1, b1, W2, b2):
    meta = _build_meta(graphs_size)
    partials = _sc_segment_sum(nodes_embedding, meta)
    out = pl.pallas_call(
        _reduce_mlp_kernel,
        out_shape=jax.ShapeDtypeStruct((_G, b2.shape[0]), jnp.float32),
    )(partials, W1, b1.reshape(1, -1), W2, b2.reshape(1, -1))
    return out
